# Initial kernel scaffold; baseline (speedup 1.0000x reference)
import jax
import jax.numpy as jnp
from jax.experimental import pallas as pl


def kernel(inputs, embeddings):
    raise NotImplementedError("write your pallas kernel here")


# speedup vs baseline: 1.5732x; 1.5732x over previous
import functools

import jax
import jax.numpy as jnp
from jax import lax
from jax.experimental import pallas as pl
from jax.experimental.pallas import tpu as pltpu
from jax.experimental.pallas import tpu_sc as plsc

_INPUT_DIM = 1000000
_OUTPUT_DIM = 32
_BATCH = 16384
_N_FIELDS = 26

_NB = _BATCH * _N_FIELDS
_NC, _NS = 2, 16
_NW = _NC * _NS
_BPW = _NB // _NW
_STREAM = 128
_CHUNK = 512
_KPC = _CHUNK // _STREAM
_NCHUNK = _BPW // _CHUNK


@functools.partial(
    pl.kernel,
    out_type=jax.ShapeDtypeStruct((_NB, _OUTPUT_DIM), jnp.float32),
    mesh=plsc.VectorSubcoreMesh(core_axis_name="c", subcore_axis_name="s"),
    compiler_params=pltpu.CompilerParams(use_tc_tiling_on_sc=False),
    scratch_types=[
        pltpu.VMEM((_BPW,), jnp.int32),
        pltpu.VMEM((_CHUNK, _OUTPUT_DIM), jnp.float32),
        pltpu.VMEM((_CHUNK, _OUTPUT_DIM), jnp.float32),
        pltpu.SemaphoreType.DMA,
        pltpu.SemaphoreType.DMA,
        pltpu.SemaphoreType.DMA,
        pltpu.SemaphoreType.DMA,
    ],
)
def _emb_lookup(table_hbm, idx_hbm, out_hbm, idx_v, rows0, rows1,
                g0, g1, w0, w1):
    wid = lax.axis_index("s") * _NC + lax.axis_index("c")
    base = wid * _BPW
    pltpu.sync_copy(idx_hbm.at[pl.ds(base, _BPW)], idx_v)

    rows = (rows0, rows1)
    gsem = (g0, g1)
    wsem = (w0, w1)

    def fire_gather(c, b):
        for j in range(_KPC):
            src = table_hbm.at[idx_v.at[pl.ds(c * _CHUNK + j * _STREAM,
                                              _STREAM)]]
            dst = rows[b].at[pl.ds(j * _STREAM, _STREAM)]
            pltpu.make_async_copy(src, dst, gsem[b]).start()

    def drain_gather(b):
        pltpu.make_async_copy(table_hbm.at[pl.ds(0, _CHUNK)], rows[b],
                              gsem[b]).wait()

    def fire_write(c, b):
        pltpu.make_async_copy(
            rows[b], out_hbm.at[pl.ds(base + c * _CHUNK, _CHUNK)],
            wsem[b]).start()

    def drain_write(b):
        pltpu.make_async_copy(rows[b], out_hbm.at[pl.ds(0, _CHUNK)],
                              wsem[b]).wait()

    fire_gather(0, 0)
    fire_gather(1, 1)

    def loop_body(i, carry):
        for b in range(2):
            c = 2 * i + b
            drain_gather(b)
            fire_write(c, b)
            drain_write(b)
            fire_gather(c + 2, b)
        return carry

    lax.fori_loop(0, (_NCHUNK - 2) // 2, loop_body, 0)

    for b in range(2):
        drain_gather(b)
        fire_write(_NCHUNK - 2 + b, b)
        drain_write(b)


def kernel(inputs, embeddings):
    idx = inputs.reshape(-1).astype(jnp.int32)
    out = _emb_lookup(embeddings, idx)
    return out.reshape(_BATCH, _N_FIELDS, _OUTPUT_DIM)


# speedup vs baseline: 1.5736x; 1.0003x over previous
import functools

import jax
import jax.numpy as jnp
from jax import lax
from jax.experimental import pallas as pl
from jax.experimental.pallas import tpu as pltpu
from jax.experimental.pallas import tpu_sc as plsc

_INPUT_DIM = 1000000
_OUTPUT_DIM = 32
_BATCH = 16384
_N_FIELDS = 26

_NB = _BATCH * _N_FIELDS
_NC, _NS = 2, 16
_NW = _NC * _NS
_BPW = _NB // _NW
_STREAM = 512
_CHUNK = 512
_KPC = _CHUNK // _STREAM
_NCHUNK = _BPW // _CHUNK


@functools.partial(
    pl.kernel,
    out_type=jax.ShapeDtypeStruct((_NB, _OUTPUT_DIM), jnp.float32),
    mesh=plsc.VectorSubcoreMesh(core_axis_name="c", subcore_axis_name="s"),
    compiler_params=pltpu.CompilerParams(use_tc_tiling_on_sc=False),
    scratch_types=[
        pltpu.VMEM((_BPW,), jnp.int32),
        pltpu.VMEM((_CHUNK, _OUTPUT_DIM), jnp.float32),
        pltpu.VMEM((_CHUNK, _OUTPUT_DIM), jnp.float32),
        pltpu.SemaphoreType.DMA,
        pltpu.SemaphoreType.DMA,
        pltpu.SemaphoreType.DMA,
        pltpu.SemaphoreType.DMA,
    ],
)
def _emb_lookup(table_hbm, idx_hbm, out_hbm, idx_v, rows0, rows1,
                g0, g1, w0, w1):
    wid = lax.axis_index("s") * _NC + lax.axis_index("c")
    base = wid * _BPW
    pltpu.sync_copy(idx_hbm.at[pl.ds(base, _BPW)], idx_v)

    rows = (rows0, rows1)
    gsem = (g0, g1)
    wsem = (w0, w1)

    def fire_gather(c, b):
        for j in range(_KPC):
            src = table_hbm.at[idx_v.at[pl.ds(c * _CHUNK + j * _STREAM,
                                              _STREAM)]]
            dst = rows[b].at[pl.ds(j * _STREAM, _STREAM)]
            pltpu.make_async_copy(src, dst, gsem[b]).start()

    def drain_gather(b):
        pltpu.make_async_copy(table_hbm.at[pl.ds(0, _CHUNK)], rows[b],
                              gsem[b]).wait()

    def fire_write(c, b):
        pltpu.make_async_copy(
            rows[b], out_hbm.at[pl.ds(base + c * _CHUNK, _CHUNK)],
            wsem[b]).start()

    def drain_write(b):
        pltpu.make_async_copy(rows[b], out_hbm.at[pl.ds(0, _CHUNK)],
                              wsem[b]).wait()

    fire_gather(0, 0)
    fire_gather(1, 1)

    def loop_body(i, carry):
        for b in range(2):
            c = 2 * i + b
            drain_gather(b)
            fire_write(c, b)
            drain_write(b)
            fire_gather(c + 2, b)
        return carry

    lax.fori_loop(0, (_NCHUNK - 2) // 2, loop_body, 0)

    for b in range(2):
        drain_gather(b)
        fire_write(_NCHUNK - 2 + b, b)
        drain_write(b)


def kernel(inputs, embeddings):
    idx = inputs.reshape(-1).astype(jnp.int32)
    out = _emb_lookup(embeddings, idx)
    return out.reshape(_BATCH, _N_FIELDS, _OUTPUT_DIM)
